# speedup vs baseline: 8.2694x; 8.2694x over previous
import functools

import jax
import jax.numpy as jnp
from jax import lax
from jax.experimental import pallas as pl
from jax.experimental.pallas import tpu as pltpu
from jax.experimental.pallas import tpu_sc as plsc

VOCAB = 100000
DIM = 128
BATCH = 4096
SEQ = 50

NC = 2
NS = 16
NW = NC * NS
B_PER_W = BATCH // NW
CHUNK = 8
ROWS = CHUNK * SEQ
NCHUNK = B_PER_W // CHUNK
LANES = 16
NV = DIM // LANES


def _sc_gather_sum(idx_flat, table):
    mesh = plsc.VectorSubcoreMesh(core_axis_name="c", subcore_axis_name="s")

    @functools.partial(
        pl.kernel,
        mesh=mesh,
        out_type=jax.ShapeDtypeStruct((BATCH, DIM), jnp.float32),
        scratch_types=[
            pltpu.VMEM((B_PER_W * SEQ,), jnp.int32),
            pltpu.VMEM((ROWS, DIM), jnp.float32),
            pltpu.VMEM((CHUNK, DIM), jnp.float32),
            pltpu.SemaphoreType.DMA,
        ],
    )
    def k(idx_hbm, table_hbm, out_hbm, idx_v, rows_v, acc_v, sem):
        wid = lax.axis_index("s") * NC + lax.axis_index("c")
        ibase = wid * (B_PER_W * SEQ)
        pltpu.sync_copy(idx_hbm.at[pl.ds(ibase, B_PER_W * SEQ)], idx_v)

        def chunk_body(c, carry):
            off = pl.multiple_of(c * ROWS, 8)
            cps = []
            for lo, sz in ((0, 128), (128, 128), (256, 128), (384, 16)):
                cps.append(pltpu.async_copy(
                    table_hbm.at[idx_v.at[pl.ds(off + lo, sz)]],
                    rows_v.at[pl.ds(lo, sz)],
                    sem,
                ))
            for cp in cps:
                cp.wait()
            for b in range(CHUNK):
                def s_body(s, accs, b=b):
                    r = b * SEQ + s
                    return tuple(a + rows_v[r, pl.ds(LANES * v, LANES)]
                                 for v, a in enumerate(accs))
                accs = lax.fori_loop(
                    0, SEQ, s_body,
                    tuple(jnp.zeros((LANES,), jnp.float32) for _ in range(NV)))
                for v, a in enumerate(accs):
                    acc_v[b, pl.ds(LANES * v, LANES)] = a
            obase = pl.multiple_of(wid * B_PER_W + c * CHUNK, 8)
            pltpu.sync_copy(acc_v, out_hbm.at[pl.ds(obase, CHUNK)])
            return carry

        lax.fori_loop(0, NCHUNK, chunk_body, 0)

    return k(idx_flat, table)


def _mlp_body(s_ref, w1_ref, b1_ref, w2_ref, b2_ref, w3_ref, b3_ref, o_ref):
    x = s_ref[...] * (1.0 / SEQ)
    h = jnp.tanh(jnp.dot(x, w1_ref[...],
                         preferred_element_type=jnp.float32) + b1_ref[...])
    h = jnp.tanh(jnp.dot(h, w2_ref[...],
                         preferred_element_type=jnp.float32) + b2_ref[...])
    z = jnp.tanh(jnp.dot(h, w3_ref[...],
                         preferred_element_type=jnp.float32) + b3_ref[...])
    a = z[:, 0:1]
    b = z[:, 1:2]
    lse = jnp.logaddexp(a, b)
    o_ref[...] = jnp.concatenate([a - lse, b - lse], axis=1)


def _tc_mlp(sums, w1, b1, w2, b2, w3, b3):
    blk = 512
    grid = BATCH // blk
    return pl.pallas_call(
        _mlp_body,
        grid=(grid,),
        in_specs=[
            pl.BlockSpec((blk, DIM), lambda i: (i, 0)),
            pl.BlockSpec((DIM, DIM), lambda i: (0, 0)),
            pl.BlockSpec((1, DIM), lambda i: (0, 0)),
            pl.BlockSpec((DIM, DIM), lambda i: (0, 0)),
            pl.BlockSpec((1, DIM), lambda i: (0, 0)),
            pl.BlockSpec((DIM, DIM), lambda i: (0, 0)),
            pl.BlockSpec((1, DIM), lambda i: (0, 0)),
        ],
        out_specs=pl.BlockSpec((blk, 2), lambda i: (i, 0)),
        out_shape=jax.ShapeDtypeStruct((BATCH, 2), jnp.float32),
    )(sums, w1, b1, w2, b2, w3, b3)


def kernel(input, emb_weight, out_w, out_b, out1_w, out1_b, out2_w, out2_b):
    idx_flat = input.reshape(-1)
    sums = _sc_gather_sum(idx_flat, emb_weight)

    w1 = jnp.zeros((DIM, DIM), jnp.float32).at[:, :25].set(out_w.T)
    b1 = jnp.zeros((1, DIM), jnp.float32).at[0, :25].set(out_b)
    w2 = jnp.zeros((DIM, DIM), jnp.float32).at[:25, :25].set(out1_w.T)
    b2 = jnp.zeros((1, DIM), jnp.float32).at[0, :25].set(out1_b)
    w3 = jnp.zeros((DIM, DIM), jnp.float32).at[:25, :2].set(out2_w.T)
    b3 = jnp.zeros((1, DIM), jnp.float32).at[0, :2].set(out2_b)

    return _tc_mlp(sums, w1, b1, w2, b2, w3, b3)


# speedup vs baseline: 11.7326x; 1.4188x over previous
import functools

import jax
import jax.numpy as jnp
from jax import lax
from jax.experimental import pallas as pl
from jax.experimental.pallas import tpu as pltpu
from jax.experimental.pallas import tpu_sc as plsc

VOCAB = 100000
DIM = 128
BATCH = 4096
SEQ = 50

NC = 2
NS = 16
NW = NC * NS
B_PER_W = BATCH // NW
CHUNK = 8
ROWS = CHUNK * SEQ
NCHUNK = B_PER_W // CHUNK
LANES = 16
NV = DIM // LANES


def _sc_gather_sum(idx_flat, table):
    mesh = plsc.VectorSubcoreMesh(core_axis_name="c", subcore_axis_name="s")

    @functools.partial(
        pl.kernel,
        mesh=mesh,
        out_type=jax.ShapeDtypeStruct((BATCH, DIM), jnp.float32),
        scratch_types=[
            pltpu.VMEM((B_PER_W * SEQ,), jnp.int32),
            pltpu.VMEM((ROWS, DIM), jnp.float32),
            pltpu.VMEM((ROWS, DIM), jnp.float32),
            pltpu.VMEM((CHUNK, DIM), jnp.float32),
            pltpu.SemaphoreType.DMA,
            pltpu.SemaphoreType.DMA,
        ],
    )
    def k(idx_hbm, table_hbm, out_hbm, idx_v, rows0, rows1, acc_v,
          sem0, sem1):
        wid = lax.axis_index("s") * NC + lax.axis_index("c")
        ibase = wid * (B_PER_W * SEQ)
        pltpu.sync_copy(idx_hbm.at[pl.ds(ibase, B_PER_W * SEQ)], idx_v)

        parts = ((0, 128), (128, 128), (256, 128), (384, 16))

        def issue(c, buf, sem):
            off = pl.multiple_of(c * ROWS, 8)
            for lo, sz in parts:
                pltpu.async_copy(
                    table_hbm.at[idx_v.at[pl.ds(off + lo, sz)]],
                    buf.at[pl.ds(lo, sz)], sem)

        def drain(buf, sem):
            for lo, sz in parts:
                pltpu.make_async_copy(
                    table_hbm.at[idx_v.at[pl.ds(lo, sz)]],
                    buf.at[pl.ds(lo, sz)], sem).wait()

        def compute(c, buf):
            for b in range(CHUNK):
                def s_body(s, accs, b=b):
                    r = b * SEQ + s
                    return tuple(a + buf[r, pl.ds(LANES * v, LANES)]
                                 for v, a in enumerate(accs))
                accs = lax.fori_loop(
                    0, SEQ, s_body,
                    tuple(jnp.zeros((LANES,), jnp.float32)
                          for _ in range(NV)),
                    unroll=2)
                for v, a in enumerate(accs):
                    acc_v[b, pl.ds(LANES * v, LANES)] = a
            obase = pl.multiple_of(wid * B_PER_W + c * CHUNK, 8)
            pltpu.sync_copy(acc_v, out_hbm.at[pl.ds(obase, CHUNK)])

        issue(0, rows0, sem0)

        def pair_body(i, carry):
            c0 = i * 2
            c1 = c0 + 1
            issue(c1, rows1, sem1)
            drain(rows0, sem0)
            compute(c0, rows0)

            @pl.when(i + 1 < NCHUNK // 2)
            def _():
                issue(c1 + 1, rows0, sem0)

            drain(rows1, sem1)
            compute(c1, rows1)
            return carry

        lax.fori_loop(0, NCHUNK // 2, pair_body, 0)

    return k(idx_flat, table)


def _mlp_body(s_ref, w1_ref, b1_ref, w2_ref, b2_ref, w3_ref, b3_ref, o_ref):
    x = s_ref[...] * (1.0 / SEQ)
    h = jnp.tanh(jnp.dot(x, w1_ref[...],
                         preferred_element_type=jnp.float32) + b1_ref[...])
    h = jnp.tanh(jnp.dot(h, w2_ref[...],
                         preferred_element_type=jnp.float32) + b2_ref[...])
    z = jnp.tanh(jnp.dot(h, w3_ref[...],
                         preferred_element_type=jnp.float32) + b3_ref[...])
    a = z[:, 0:1]
    b = z[:, 1:2]
    lse = jnp.logaddexp(a, b)
    o_ref[...] = jnp.concatenate([a - lse, b - lse], axis=1)


def _tc_mlp(sums, w1, b1, w2, b2, w3, b3):
    blk = 512
    grid = BATCH // blk
    return pl.pallas_call(
        _mlp_body,
        grid=(grid,),
        in_specs=[
            pl.BlockSpec((blk, DIM), lambda i: (i, 0)),
            pl.BlockSpec((DIM, DIM), lambda i: (0, 0)),
            pl.BlockSpec((1, DIM), lambda i: (0, 0)),
            pl.BlockSpec((DIM, DIM), lambda i: (0, 0)),
            pl.BlockSpec((1, DIM), lambda i: (0, 0)),
            pl.BlockSpec((DIM, DIM), lambda i: (0, 0)),
            pl.BlockSpec((1, DIM), lambda i: (0, 0)),
        ],
        out_specs=pl.BlockSpec((blk, 2), lambda i: (i, 0)),
        out_shape=jax.ShapeDtypeStruct((BATCH, 2), jnp.float32),
    )(sums, w1, b1, w2, b2, w3, b3)


def kernel(input, emb_weight, out_w, out_b, out1_w, out1_b, out2_w, out2_b):
    idx_flat = input.reshape(-1)
    sums = _sc_gather_sum(idx_flat, emb_weight)

    w1 = jnp.zeros((DIM, DIM), jnp.float32).at[:, :25].set(out_w.T)
    b1 = jnp.zeros((1, DIM), jnp.float32).at[0, :25].set(out_b)
    w2 = jnp.zeros((DIM, DIM), jnp.float32).at[:25, :25].set(out1_w.T)
    b2 = jnp.zeros((1, DIM), jnp.float32).at[0, :25].set(out1_b)
    w3 = jnp.zeros((DIM, DIM), jnp.float32).at[:25, :2].set(out2_w.T)
    b3 = jnp.zeros((1, DIM), jnp.float32).at[0, :2].set(out2_b)

    return _tc_mlp(sums, w1, b1, w2, b2, w3, b3)
